# Initial kernel scaffold; baseline (speedup 1.0000x reference)
import jax
import jax.numpy as jnp
from jax.experimental import pallas as pl


def kernel(idx, targets, table):
    raise NotImplementedError("write your pallas kernel here")


# speedup vs baseline: 1.3663x; 1.3663x over previous
import functools

import jax
import jax.numpy as jnp
from jax import lax
from jax.experimental import pallas as pl
from jax.experimental.pallas import tpu as pltpu
from jax.experimental.pallas import tpu_sc as plsc

VOCAB = 1000
LSE_PAD = 1024
BATCH, SEQ = 1024, 50
NTOK = BATCH * SEQ
NC, NS = 2, 16
NW = NC * NS
TPW = NTOK // NW
CHUNK = 32
NCHUNK = TPW // CHUNK


def _lse_body(tab_ref, out_ref):
    x = tab_ref[...]
    m = jnp.max(x, axis=1, keepdims=True)
    s = jnp.sum(jnp.exp(x - m), axis=1, keepdims=True)
    out_ref[...] = m + jnp.log(s)


_lse_call = pl.pallas_call(
    _lse_body,
    out_shape=jax.ShapeDtypeStruct((VOCAB, 1), jnp.float32),
)


def _sc_body(table_hbm, idx_hbm, tgt_hbm, lse_hbm, out_hbm, part_hbm,
             idx_v, tgt_v, lse_v, rows_v, acc_v, gsem):
    c = lax.axis_index("c")
    s = lax.axis_index("s")
    wid = s * NC + c
    base = wid * TPW

    pltpu.sync_copy(idx_hbm.at[wid], idx_v)
    pltpu.sync_copy(tgt_hbm.at[wid], tgt_v)
    pltpu.sync_copy(lse_hbm, lse_v)

    def chunk_body(g, acc):
        pltpu.async_copy(table_hbm.at[idx_v.at[g]], rows_v, gsem).wait()
        for h in range(CHUNK // 16):
            rid = lax.iota(jnp.int32, 16) + (h * 16)
            tg = tgt_v[g, pl.ds(h * 16, 16)]
            pk = plsc.load_gather(rows_v, [rid, tg])
            ix = idx_v[g, pl.ds(h * 16, 16)]
            ls = plsc.load_gather(lse_v, [ix])
            acc = acc + (ls - pk)
        pltpu.sync_copy(rows_v, out_hbm.at[pl.ds(base + g * CHUNK, CHUNK)])
        return acc

    acc = lax.fori_loop(0, NCHUNK, chunk_body,
                        jnp.zeros((16,), jnp.float32))
    acc_v[...] = acc
    pltpu.sync_copy(acc_v, part_hbm.at[wid])


_sc_call = functools.partial(
    pl.kernel,
    mesh=plsc.VectorSubcoreMesh(core_axis_name="c", subcore_axis_name="s"),
    compiler_params=pltpu.CompilerParams(use_tc_tiling_on_sc=False,
                                         needs_layout_passes=False),
    out_type=[
        jax.ShapeDtypeStruct((NTOK, VOCAB), jnp.float32),
        jax.ShapeDtypeStruct((NW, 16), jnp.float32),
    ],
    scratch_types=[
        pltpu.VMEM((NCHUNK, CHUNK), jnp.int32),
        pltpu.VMEM((NCHUNK, CHUNK), jnp.int32),
        pltpu.VMEM((LSE_PAD,), jnp.float32),
        pltpu.VMEM((CHUNK, VOCAB), jnp.float32),
        pltpu.VMEM((16,), jnp.float32),
        pltpu.SemaphoreType.DMA,
    ],
)(_sc_body)


def _loss_body(part_ref, out_ref):
    out_ref[...] = jnp.sum(part_ref[...], keepdims=True).reshape(1, 1) * (
        1.0 / NTOK)


_loss_call = pl.pallas_call(
    _loss_body,
    out_shape=jax.ShapeDtypeStruct((1, 1), jnp.float32),
)


@jax.jit
def kernel(idx, targets, table):
    idx3 = idx.astype(jnp.int32).reshape(NW, NCHUNK, CHUNK)
    tgt3 = targets.astype(jnp.int32).reshape(NW, NCHUNK, CHUNK)
    lse = _lse_call(table)
    lse_p = jnp.pad(lse.reshape(-1), (0, LSE_PAD - VOCAB))
    out_flat, parts = _sc_call(table, idx3, tgt3, lse_p)
    loss = _loss_call(parts)[0, 0]
    return out_flat.reshape(BATCH, SEQ, VOCAB), loss


# speedup vs baseline: 1.4310x; 1.0474x over previous
import functools

import jax
import jax.numpy as jnp
from jax import lax
from jax.experimental import pallas as pl
from jax.experimental.pallas import tpu as pltpu
from jax.experimental.pallas import tpu_sc as plsc

VOCAB = 1000
LSE_PAD = 1024
BATCH, SEQ = 1024, 50
NTOK = BATCH * SEQ
NC, NS = 2, 16
NW = NC * NS
TPW = NTOK // NW
CHUNK = 32
NCHUNK = TPW // CHUNK


def _lse_body(tab_ref, out_ref):
    x = tab_ref[...]
    m = jnp.max(x, axis=1, keepdims=True)
    s = jnp.sum(jnp.exp(x - m), axis=1, keepdims=True)
    out_ref[...] = m + jnp.log(s)


_lse_call = pl.pallas_call(
    _lse_body,
    out_shape=jax.ShapeDtypeStruct((VOCAB, 1), jnp.float32),
)


def _sc_body(table_hbm, idx_hbm, tgt_hbm, lse_hbm, out_hbm, part_hbm,
             idx_v, tgt_v, lse_v, rows_v, acc_v,
             gsem0, gsem1, wsem0, wsem1):
    c = lax.axis_index("c")
    s = lax.axis_index("s")
    wid = s * NC + c
    base = wid * TPW

    pltpu.sync_copy(idx_hbm.at[wid], idx_v)
    pltpu.sync_copy(tgt_hbm.at[wid], tgt_v)
    pltpu.sync_copy(lse_hbm, lse_v)

    gsems = [gsem0, gsem1]
    wsems = [wsem0, wsem1]

    def gather(g, slot, sem):
        return pltpu.make_async_copy(table_hbm.at[idx_v.at[g]],
                                     rows_v.at[slot], sem)

    def write(g, slot, sem):
        return pltpu.make_async_copy(
            rows_v.at[slot], out_hbm.at[pl.ds(base + g * CHUNK, CHUNK)], sem)

    gather(0, 0, gsem0).start()

    def pair_body(t, acc):
        for b in range(2):
            g = t * 2 + b
            other = 1 - b

            @pl.when(g + 1 < NCHUNK)
            def _():
                @pl.when(g >= 1)
                def _():
                    write(g - 1, other, wsems[other]).wait()
                gather(g + 1, other, gsems[other]).start()

            gather(g, b, gsems[b]).wait()
            for h in range(CHUNK // 16):
                rid = lax.iota(jnp.int32, 16) + (h * 16)
                tg = tgt_v[g, pl.ds(h * 16, 16)]
                pk = plsc.load_gather(rows_v.at[b], [rid, tg])
                ix = idx_v[g, pl.ds(h * 16, 16)]
                ls = plsc.load_gather(lse_v, [ix])
                acc = acc + (ls - pk)
            write(g, b, wsems[b]).start()
        return acc

    acc = lax.fori_loop(0, NCHUNK // 2, pair_body,
                        jnp.zeros((16,), jnp.float32))
    write(NCHUNK - 2, 0, wsem0).wait()
    write(NCHUNK - 1, 1, wsem1).wait()
    acc_v[...] = acc
    pltpu.sync_copy(acc_v, part_hbm.at[wid])


_sc_call = functools.partial(
    pl.kernel,
    mesh=plsc.VectorSubcoreMesh(core_axis_name="c", subcore_axis_name="s"),
    compiler_params=pltpu.CompilerParams(use_tc_tiling_on_sc=False,
                                         needs_layout_passes=False),
    out_type=[
        jax.ShapeDtypeStruct((NTOK, VOCAB), jnp.float32),
        jax.ShapeDtypeStruct((NW, 16), jnp.float32),
    ],
    scratch_types=[
        pltpu.VMEM((NCHUNK, CHUNK), jnp.int32),
        pltpu.VMEM((NCHUNK, CHUNK), jnp.int32),
        pltpu.VMEM((LSE_PAD,), jnp.float32),
        pltpu.VMEM((2, CHUNK, VOCAB), jnp.float32),
        pltpu.VMEM((16,), jnp.float32),
        pltpu.SemaphoreType.DMA,
        pltpu.SemaphoreType.DMA,
        pltpu.SemaphoreType.DMA,
        pltpu.SemaphoreType.DMA,
    ],
)(_sc_body)


def _loss_body(part_ref, out_ref):
    out_ref[...] = jnp.sum(part_ref[...], keepdims=True).reshape(1, 1) * (
        1.0 / NTOK)


_loss_call = pl.pallas_call(
    _loss_body,
    out_shape=jax.ShapeDtypeStruct((1, 1), jnp.float32),
)


@jax.jit
def kernel(idx, targets, table):
    idx3 = idx.astype(jnp.int32).reshape(NW, NCHUNK, CHUNK)
    tgt3 = targets.astype(jnp.int32).reshape(NW, NCHUNK, CHUNK)
    lse = _lse_call(table)
    lse_p = jnp.pad(lse.reshape(-1), (0, LSE_PAD - VOCAB))
    out_flat, parts = _sc_call(table, idx3, tgt3, lse_p)
    loss = _loss_call(parts)[0, 0]
    return out_flat.reshape(BATCH, SEQ, VOCAB), loss


# speedup vs baseline: 2.7130x; 1.8959x over previous
import functools

import jax
import jax.numpy as jnp
from jax import lax
from jax.experimental import pallas as pl
from jax.experimental.pallas import tpu as pltpu
from jax.experimental.pallas import tpu_sc as plsc

VOCAB = 1000
VPAD = 1024
LSE_PAD = 1024
BATCH, SEQ = 1024, 50
NTOK = BATCH * SEQ
NC, NS = 2, 16
NW = NC * NS
BPW = BATCH // NW
TPW = NTOK // NW
LCH = 80
NLCH = TPW // LCH


def _lse_body(tab_ref, out_ref):
    x = tab_ref[...]
    m = jnp.max(x, axis=1, keepdims=True)
    s = jnp.sum(jnp.exp(x - m), axis=1, keepdims=True)
    out_ref[...] = m + jnp.log(s)


_lse_call = pl.pallas_call(
    _lse_body,
    out_shape=jax.ShapeDtypeStruct((VOCAB, 1), jnp.float32),
)


def _gather_body(table_hbm, idx_hbm, out_hbm, idx_v, rows_v,
                 gsem0, gsem1, wsem0, wsem1):
    c = lax.axis_index("c")
    s = lax.axis_index("s")
    wid = s * NC + c
    base = wid * BPW

    pltpu.sync_copy(idx_hbm.at[pl.ds(wid * (BPW * 64), BPW * 64)], idx_v)

    gsems = [gsem0, gsem1]
    wsems = [wsem0, wsem1]

    def gather(g, slot, sem):
        return pltpu.make_async_copy(
            table_hbm.at[idx_v.at[pl.ds(g * 64, SEQ)]], rows_v.at[slot], sem)

    def write(g, slot, sem):
        return pltpu.make_async_copy(rows_v.at[slot], out_hbm.at[base + g],
                                     sem)

    gather(0, 0, gsem0).start()

    def pair_body(t, _):
        for b in range(2):
            g = t * 2 + b
            other = 1 - b
            @pl.when(g + 1 < BPW)
            def _():
                @pl.when(g >= 1)
                def _():
                    write(g - 1, other, wsems[other]).wait()
                gather(g + 1, other, gsems[other]).start()
            gather(g, b, gsems[b]).wait()
            write(g, b, wsems[b]).start()
        return 0

    lax.fori_loop(0, BPW // 2, pair_body, 0)
    write(BPW - 2, 0, wsem0).wait()
    write(BPW - 1, 1, wsem1).wait()


_gather_call = functools.partial(
    pl.kernel,
    mesh=plsc.VectorSubcoreMesh(core_axis_name="c", subcore_axis_name="s"),
    compiler_params=pltpu.CompilerParams(use_tc_tiling_on_sc=True,
                                         needs_layout_passes=False),
    out_type=jax.ShapeDtypeStruct((BATCH, SEQ, VPAD), jnp.float32),
    scratch_types=[
        pltpu.VMEM((BPW * 64,), jnp.int32),
        pltpu.VMEM((2, SEQ, VPAD), jnp.float32),
        pltpu.SemaphoreType.DMA,
        pltpu.SemaphoreType.DMA,
        pltpu.SemaphoreType.DMA,
        pltpu.SemaphoreType.DMA,
    ],
)(_gather_body)


def _loss_body(tabflat_hbm, idxflat_hbm, pickflat_hbm, lse_hbm, part_hbm,
               idx_v, pick_v, lse_v, pkv_v, acc_v, sem):
    c = lax.axis_index("c")
    s = lax.axis_index("s")
    wid = s * NC + c
    base = wid * TPW

    pltpu.sync_copy(idxflat_hbm.at[pl.ds(base, TPW)], idx_v)
    pltpu.sync_copy(pickflat_hbm.at[pl.ds(base, TPW)], pick_v)
    pltpu.sync_copy(lse_hbm, lse_v)

    def chunk_body(g, acc):
        pltpu.async_copy(tabflat_hbm.at[pick_v.at[pl.ds(g * LCH, LCH)]],
                         pkv_v, sem).wait()
        for h in range(LCH // 16):
            ixh = idx_v[pl.ds(g * LCH + h * 16, 16)]
            ls = plsc.load_gather(lse_v, [ixh])
            acc = acc + (ls - pkv_v[pl.ds(h * 16, 16)])
        return acc

    acc = lax.fori_loop(0, NLCH, chunk_body, jnp.zeros((16,), jnp.float32))
    acc_v[...] = acc
    pltpu.sync_copy(acc_v, part_hbm.at[wid])


_loss_call = functools.partial(
    pl.kernel,
    mesh=plsc.VectorSubcoreMesh(core_axis_name="c", subcore_axis_name="s"),
    compiler_params=pltpu.CompilerParams(use_tc_tiling_on_sc=False,
                                         needs_layout_passes=False),
    out_type=jax.ShapeDtypeStruct((NW, 16), jnp.float32),
    scratch_types=[
        pltpu.VMEM((TPW,), jnp.int32),
        pltpu.VMEM((TPW,), jnp.int32),
        pltpu.VMEM((LSE_PAD,), jnp.float32),
        pltpu.VMEM((LCH,), jnp.float32),
        pltpu.VMEM((16,), jnp.float32),
        pltpu.SemaphoreType.DMA,
    ],
)(_loss_body)


def _reduce_body(part_ref, out_ref):
    out_ref[...] = jnp.sum(part_ref[...], keepdims=True).reshape(1, 1) * (
        1.0 / NTOK)


_reduce_call = pl.pallas_call(
    _reduce_body,
    out_shape=jax.ShapeDtypeStruct((1, 1), jnp.float32),
)


@jax.jit
def kernel(idx, targets, table):
    idx32 = idx.astype(jnp.int32)
    tgt32 = targets.astype(jnp.int32)
    table_p = jnp.pad(table, ((0, 0), (0, VPAD - VOCAB)))
    idxp1 = jnp.pad(idx32, ((0, 0), (0, 64 - SEQ))).reshape(-1)
    idxflat = idx32.reshape(-1)
    pickflat = idxflat * VPAD + tgt32.reshape(-1)
    lse = _lse_call(table)
    lse_p = jnp.pad(lse.reshape(-1), (0, LSE_PAD - VOCAB))
    logits = _gather_call(table_p, idxp1)[:, :, :VOCAB]
    parts = _loss_call(table_p.reshape(-1), idxflat, pickflat, lse_p)
    loss = _reduce_call(parts)[0, 0]
    return logits, loss
